# baseline (speedup 1.0000x reference)
import functools

import jax
import jax.numpy as jnp
from jax import lax
from jax.experimental import pallas as pl
from jax.experimental.pallas import tpu as pltpu
from jax.experimental.pallas import tpu_sc as plsc

B, S, D = 4, 4096, 4096


def _fin_body(lg_ref, sot_ref, lab_ref, b_ref, loss_ref, pred_ref):
    lg = lg_ref[...] + b_ref[...]
    mask = sot_ref[...] != 0
    neg_inf = jnp.float32(-jnp.inf)
    ml = jnp.where(mask, lg, neg_inf)
    m = jnp.max(ml, axis=1, keepdims=True)
    m_safe = jnp.where(m == neg_inf, jnp.float32(0.0), m)
    su = jnp.sum(jnp.exp(ml - m_safe), axis=1, keepdims=True)
    lse = m_safe + jnp.log(su)

    iota = lax.broadcasted_iota(jnp.int32, (B, S), 1)
    big = jnp.int32(1 << 30)
    hit = ml == m
    idx = jnp.min(jnp.where(hit, iota, big), axis=1, keepdims=True)

    mi = mask.astype(jnp.int32)
    r = mi
    sh = 1
    while sh < S:
        zeros = jnp.zeros((B, sh), jnp.int32)
        r = r + jnp.concatenate([zeros, r[:, : S - sh]], axis=1)
        sh *= 2
    rank = r - 1
    pred = jnp.sum(jnp.where(iota == idx, rank, 0), axis=1, keepdims=True)

    lab = lab_ref[...]
    sel = mask & (rank == lab)
    exists = jnp.sum(sel.astype(jnp.int32), axis=1, keepdims=True) > 0
    chosen_sel = jnp.sum(jnp.where(sel, lg, jnp.float32(0.0)), axis=1,
                         keepdims=True)
    chosen = jnp.where(exists, chosen_sel, lg[:, 0:1])
    loss = jnp.sum(lse - chosen) * jnp.float32(1.0 / B)
    loss_ref[...] = loss.reshape(1, 1)
    pred_ref[...] = pred


def _finish(lg, sot, labels, b):
    return pl.pallas_call(
        _fin_body,
        in_specs=[
            pl.BlockSpec((B, S), lambda: (0, 0)),
            pl.BlockSpec((B, S), lambda: (0, 0)),
            pl.BlockSpec((B, 1), lambda: (0, 0)),
            pl.BlockSpec((1, 1), lambda: (0, 0)),
        ],
        out_specs=(
            pl.BlockSpec((1, 1), lambda: (0, 0)),
            pl.BlockSpec((B, 1), lambda: (0, 0)),
        ),
        out_shape=(
            jax.ShapeDtypeStruct((1, 1), jnp.float32),
            jax.ShapeDtypeStruct((B, 1), jnp.int32),
        ),
    )(lg, sot, labels.reshape(B, 1), b.reshape(1, 1))


_NC, _NS, _L = 2, 16, 16
_NW = _NC * _NS
_CHUNK = 8


def _sc_logits(x2d, sot_flat, w):
    nrows = B * S
    slots = nrows // _NW
    ngrp = slots // _L
    mesh = plsc.VectorSubcoreMesh(core_axis_name="c", subcore_axis_name="s")

    @functools.partial(
        pl.kernel,
        out_type=jax.ShapeDtypeStruct((nrows,), jnp.float32),
        mesh=mesh,
        compiler_params=pltpu.CompilerParams(needs_layout_passes=False),
        scratch_types=[
            pltpu.VMEM((slots,), jnp.int32),
            pltpu.VMEM((slots,), jnp.int32),
            pltpu.VMEM((D,), jnp.float32),
            pltpu.VMEM((_CHUNK, D), jnp.float32),
            pltpu.VMEM((_CHUNK, D), jnp.float32),
            pltpu.VMEM((slots,), jnp.float32),
            pltpu.SemaphoreType.DMA,
            pltpu.SemaphoreType.DMA,
        ],
    )
    def body(x_hbm, sot_hbm, w_hbm, out_hbm,
             mask_v, idx_v, w_v, buf0, buf1, out_v, sem0, sem1):
        wid = lax.axis_index("s") * _NC + lax.axis_index("c")
        base_glob = wid * slots
        pltpu.sync_copy(sot_hbm.at[pl.ds(base_glob, slots)], mask_v)
        pltpu.sync_copy(w_hbm, w_v)

        neg = jnp.full((_L,), -jnp.inf, jnp.float32)
        zero_i = jnp.zeros((_L,), jnp.int32)
        for i in range(ngrp):
            out_v[pl.ds(i * _L, _L)] = neg
            idx_v[pl.ds(i * _L, _L)] = zero_i

        lane = lax.iota(jnp.int32, _L)
        one_i = jnp.full((_L,), 1, jnp.int32)
        s_vec = jnp.full((_L,), S, jnp.int32)
        off = jnp.int32(0)
        for i in range(ngrp):
            m16 = mask_v[pl.ds(i * _L, _L)]
            gidx = lane + (base_glob + i * _L)
            keep = (m16 != zero_i) | (jnp.remainder(gidx, s_vec) == zero_i)
            cum = plsc.cumsum(jnp.where(keep, one_i, zero_i))
            plsc.store_scatter(idx_v, [cum - 1 + off], gidx, mask=keep)
            off = off + jnp.max(cum)
        total = off
        npairs = lax.div(total + jnp.int32(2 * _CHUNK - 1),
                         jnp.int32(2 * _CHUNK))

        @pl.when(npairs > 0)
        def _():
            pltpu.async_copy(x_hbm.at[idx_v.at[pl.ds(0, _CHUNK)]],
                             buf0, sem0)

        def compute8(buf):
            init = tuple(jnp.zeros((_L,), jnp.float32)
                         for _ in range(_CHUNK))

            def kbody(kk, accs_in):
                wk = w_v[pl.ds(kk, _L)]
                return tuple(accs_in[r] + buf[r, pl.ds(kk, _L)] * wk
                             for r in range(_CHUNK))
            accs = plsc.parallel_loop(
                0, D, step=_L, unroll=4, carry=init)(kbody)
            return [jnp.sum(a) for a in accs]

        def pair_body(gg, carry):
            g0 = gg * 2
            pltpu.async_copy(
                x_hbm.at[idx_v.at[pl.ds((g0 + 1) * _CHUNK, _CHUNK)]],
                buf1, sem1)
            pltpu.make_async_copy(
                x_hbm.at[idx_v.at[pl.ds(g0 * _CHUNK, _CHUNK)]],
                buf0, sem0).wait()
            res0 = compute8(buf0)

            @pl.when(gg + 1 < npairs)
            def _():
                pltpu.async_copy(
                    x_hbm.at[idx_v.at[pl.ds((g0 + 2) * _CHUNK, _CHUNK)]],
                    buf0, sem0)

            pltpu.make_async_copy(
                x_hbm.at[idx_v.at[pl.ds((g0 + 1) * _CHUNK, _CHUNK)]],
                buf1, sem1).wait()
            res1 = compute8(buf1)

            res = jnp.zeros((_L,), jnp.float32)
            zero_f = jnp.zeros((_L,), jnp.float32)
            for r, sc in enumerate(res0 + res1):
                rvec = jnp.full((_L,), r, jnp.int32)
                res = jnp.where(lane == rvec, zero_f + sc, res)
            slot16 = idx_v[pl.ds(gg * _L, _L)] - base_glob
            valid = (lane + gg * _L) < (zero_i + total)
            plsc.store_scatter(out_v, [slot16], res, mask=valid)
            return carry

        lax.fori_loop(0, npairs, pair_body, jnp.int32(0))
        pltpu.sync_copy(out_v, out_hbm.at[pl.ds(base_glob, slots)])

    return body(x2d, sot_flat, w)


def kernel(sequence_output, sot_positions, labels, w, b):
    x2d = sequence_output.reshape(B * S, D)
    lg = _sc_logits(x2d, sot_positions.reshape(B * S), w).reshape(B, S)
    loss, pred = _finish(lg, sot_positions, labels, b)
    return loss.reshape(()), pred.reshape(B), labels
